# baseline (speedup 1.0000x reference)
import jax
import jax.numpy as jnp
from jax import lax
from jax.experimental import pallas as pl
from jax.experimental.pallas import tpu as pltpu
from jax.experimental.pallas import tpu_sc as plsc

B = 16384
F = 128
P = 26
NM = 25
OUT_W = F + 2

NC = 2
NS = 16
NW = NC * NS
ROWS = B // NW
CHUNK = 32
NCHUNK = ROWS // CHUNK


def _sc_body(
    feat_hbm, out_hbm,
    feat_v0, feat_v1, rep_v0, rep_v1,
    sem_in, sem_out,
):
    wid = lax.axis_index("s") * NC + lax.axis_index("c")
    feat_b = (feat_v0, feat_v1)
    rep_b = (rep_v0, rep_v1)

    def stage_in(c, p):
        base = wid * ROWS + c * CHUNK
        return pltpu.async_copy(
            feat_hbm.at[pl.ds(base, CHUNK), :], feat_b[p], sem_in
        )

    def build(p):
        feat_v, rep_v = feat_b[p], rep_b[p]

        def row_body(r, carry):
            for j in range(F // 16):
                vals = feat_v[r, pl.ds(16 * j, 16)]
                for s in range(8):
                    rep_v[r, s, pl.ds(16 * j, 16)] = vals
            return carry

        lax.fori_loop(0, CHUNK, row_body, 0)

    def stage_out(c, p):
        base = wid * ROWS + c * CHUNK
        copies = [
            pltpu.async_copy(
                rep_b[p],
                out_hbm.at[pl.ds(base, CHUNK), pl.ds(8 * m, 8), pl.ds(0, F)],
                sem_out,
            )
            for m in range(3)
        ]
        copies.append(
            pltpu.async_copy(
                rep_b[p].at[:, pl.ds(7, 1), :],
                out_hbm.at[pl.ds(base, CHUNK), pl.ds(NM - 1, 1), pl.ds(0, F)],
                sem_out,
            )
        )
        return copies

    pending_in = {0: stage_in(0, 0)}
    pending_out = {}
    for c in range(NCHUNK):
        p = c % 2
        if c - 2 in pending_out:
            for d in pending_out.pop(c - 2):
                d.wait()
        pending_in.pop(c).wait()
        build(p)
        if c + 1 < NCHUNK:
            pending_in[c + 1] = stage_in(c + 1, 1 - p)
        pending_out[c] = stage_out(c, p)
    for copies in pending_out.values():
        for d in copies:
            d.wait()


def _tc_pairs_body(_, par_ref, out_ref, pair_v, sem):
    i = pl.program_id(0)
    par = par_ref[...]
    pair_v[...] = jnp.stack([par[:, 0:NM], par[:, 1 : NM + 1]], axis=-1)
    bb = par.shape[0]
    pltpu.make_async_copy(
        pair_v, out_ref.at[pl.ds(i * bb, bb), :, pl.ds(F, 2)], sem
    ).start()
    pltpu.make_async_copy(
        pair_v, out_ref.at[pl.ds(i * bb, bb), :, pl.ds(F, 2)], sem
    ).wait()


@jax.jit
def kernel(features, parameters):
    mesh = plsc.VectorSubcoreMesh(
        core_axis_name="c", subcore_axis_name="s", num_cores=NC, num_subcores=NS
    )
    sc_run = pl.kernel(
        _sc_body,
        out_type=jax.ShapeDtypeStruct((B, NM, OUT_W), jnp.float32),
        mesh=mesh,
        scratch_types=[
            pltpu.VMEM((CHUNK, F), jnp.float32),
            pltpu.VMEM((CHUNK, F), jnp.float32),
            pltpu.VMEM((CHUNK, 8, F), jnp.float32),
            pltpu.VMEM((CHUNK, 8, F), jnp.float32),
            pltpu.SemaphoreType.DMA,
            pltpu.SemaphoreType.DMA,
        ],
    )
    feat_out = sc_run(features)

    BB = 1024
    pairs_done = pl.pallas_call(
        _tc_pairs_body,
        grid=(B // BB,),
        in_specs=[
            pl.BlockSpec(memory_space=pl.ANY),
            pl.BlockSpec((BB, P), lambda i: (i, 0)),
        ],
        out_specs=pl.BlockSpec(memory_space=pl.ANY),
        out_shape=jax.ShapeDtypeStruct((B, NM, OUT_W), jnp.float32),
        scratch_shapes=[
            pltpu.VMEM((BB, NM, 2), jnp.float32),
            pltpu.SemaphoreType.DMA,
        ],
        input_output_aliases={0: 0},
    )(feat_out, parameters)
    return pairs_done


# speedup vs baseline: 6.9641x; 6.9641x over previous
import jax
import jax.numpy as jnp
from jax import lax
from jax.experimental import pallas as pl
from jax.experimental.pallas import tpu as pltpu
from jax.experimental.pallas import tpu_sc as plsc

B = 16384
F = 128
P = 26
NM = 25
OUT_W = F + 2

NC = 2
NS = 16
NW = NC * NS
COLS = B // NW


def _sc_body(ft_hbm, out_hbm, ft_v, sem_in, sem_out):
    wid = lax.axis_index("s") * NC + lax.axis_index("c")
    base = wid * COLS

    pltpu.async_copy(ft_hbm.at[:, :, pl.ds(base, COLS)], ft_v, sem_in).wait()

    copies = [
        pltpu.async_copy(
            ft_v,
            out_hbm.at[pl.ds(i, 1), pl.ds(0, F), pl.ds(base, COLS)],
            sem_out,
        )
        for i in range(NM)
    ]
    for d in copies:
        d.wait()


def _tc_pairs_body(_, pt_ref, out_ref, pair_v, sem):
    for i in range(NM):
        pair_v[i, 0, :] = pt_ref[0, i, :]
        pair_v[i, 1, :] = pt_ref[0, i + 1, :]
    pltpu.make_async_copy(pair_v, out_ref.at[:, pl.ds(F, 2), :], sem).start()
    pltpu.make_async_copy(pair_v, out_ref.at[:, pl.ds(F, 2), :], sem).wait()


@jax.jit
def kernel(features, parameters):
    ft = features.T.reshape(1, F, B)
    pt = parameters.T.reshape(1, P, B)
    mesh = plsc.VectorSubcoreMesh(
        core_axis_name="c", subcore_axis_name="s", num_cores=NC, num_subcores=NS
    )
    sc_run = pl.kernel(
        _sc_body,
        out_type=jax.ShapeDtypeStruct((NM, OUT_W, B), jnp.float32),
        mesh=mesh,
        scratch_types=[
            pltpu.VMEM((1, F, COLS), jnp.float32),
            pltpu.SemaphoreType.DMA,
            pltpu.SemaphoreType.DMA,
        ],
    )
    feat_out = sc_run(ft)

    out = pl.pallas_call(
        _tc_pairs_body,
        in_specs=[
            pl.BlockSpec(memory_space=pl.ANY),
            pl.BlockSpec((1, P, B), lambda: (0, 0, 0)),
        ],
        out_specs=pl.BlockSpec(memory_space=pl.ANY),
        out_shape=jax.ShapeDtypeStruct((NM, OUT_W, B), jnp.float32),
        scratch_shapes=[
            pltpu.VMEM((NM, 2, B), jnp.float32),
            pltpu.SemaphoreType.DMA,
        ],
        input_output_aliases={0: 0},
    )(feat_out, pt)
    return out.transpose(2, 0, 1)
